# baseline (speedup 1.0000x reference)
import functools

import jax
import jax.numpy as jnp
from jax import lax
from jax.experimental import pallas as pl
from jax.experimental.pallas import tpu as pltpu
from jax.experimental.pallas import tpu_sc as plsc

_RATIOS = (0.1, 0.25, 0.5)
_D = 1024
_NV = _D // 16
_BR = 16
_IMAX = 0xFFFFFFFF
_NW = 32
_SC_ROWS = 8704
_NBITS = 31


def _vsort(x, desc):
    k, _ = plsc.sort_key_val(x, x, descending=desc)
    return k


def _lane_min(v):
    return lax.reduce_min(v, (0,))


def _sc_body(f_hbm, b_hbm, w_hbm, o_hbm, f_v, b_v, o_v, bits_v, w_v, *,
             sc_rows):
    nc = 2
    wid = lax.axis_index("s") * nc + lax.axis_index("c")
    rows_per_w = sc_rows // _NW
    nbatch = rows_per_w // _BR
    blk = _BR * _D

    pltpu.sync_copy(w_hbm, w_v)
    w0 = w_v[pl.ds(0, 16)]
    w1 = w_v[pl.ds(16, 16)]
    w2 = w_v[pl.ds(32, 16)]

    def batch_body(t, carry):
        off = (wid * rows_per_w + t * _BR) * _D
        pltpu.sync_copy(f_hbm.at[pl.ds(off, blk)], f_v)
        pltpu.sync_copy(b_hbm.at[pl.ds(off, blk)], b_v)

        def row_body(r, carry2):
            boff = r * _D
            vs = []
            for v in range(_NV):
                fv = f_v[pl.ds(boff + v * 16, 16)]
                bv = b_v[pl.ds(boff + v * 16, 16)]
                dbits = lax.bitcast_convert_type(jnp.abs(fv - bv), jnp.uint32)
                bits_v[pl.ds(v * 16, 16)] = dbits
                vs.append(_vsort(dbits, desc=(v % 2 == 1)))
            for k in range(5, 10):
                sh = k - 4
                dv = 1 << (sh - 1)
                while dv >= 1:
                    for g in range(0, _NV, 2 * dv):
                        for i in range(g, g + dv):
                            asc = ((i >> sh) % 2 == 0)
                            a, b = vs[i], vs[i + dv]
                            lo = jnp.minimum(a, b)
                            hi = jnp.maximum(a, b)
                            vs[i], vs[i + dv] = (lo, hi) if asc else (hi, lo)
                    dv //= 2
                vs = [_vsort(x, desc=(((i >> sh) % 2) == 1))
                      for i, x in enumerate(vs)]
            u = [jnp.maximum(vs[i], vs[i + 32]) for i in range(32)]
            m = u[0]
            for x in u[1:]:
                m = jnp.minimum(m, x)
            t512 = jnp.full((16,), _lane_min(m), jnp.uint32)
            u2 = [jnp.maximum(u[i], u[i + 16]) for i in range(16)]
            m = u2[0]
            for x in u2[1:]:
                m = jnp.minimum(m, x)
            t256 = jnp.full((16,), _lane_min(m), jnp.uint32)
            u3 = [jnp.maximum(u2[i], u2[i + 8]) for i in range(8)]
            for dv in (4, 2, 1):
                for g in range(0, 8, 2 * dv):
                    for i in range(g, g + dv):
                        a, b = u3[i], u3[i + dv]
                        u3[i] = jnp.minimum(a, b)
                        u3[i + dv] = jnp.maximum(a, b)
            u3 = [_vsort(x, desc=False) for x in u3]
            lane = lax.broadcasted_iota(jnp.int32, (16,), 0)
            pick = jnp.where(lane == 10, u3[1],
                             jnp.full((16,), _IMAX, jnp.uint32))
            t102 = jnp.full((16,), _lane_min(pick), jnp.uint32)

            zero = jnp.zeros((16,), jnp.float32)
            for v in range(_NV):
                x = bits_v[pl.ds(v * 16, 16)]
                fv = f_v[pl.ds(boff + v * 16, 16)]
                wt = jnp.where(x >= t102, w0, zero)
                wt = wt + jnp.where(x >= t256, w1, zero)
                wt = wt + jnp.where(x >= t512, w2, zero)
                o_v[pl.ds(boff + v * 16, 16)] = fv * wt
            return carry2

        lax.fori_loop(0, _BR, row_body, 0)
        pltpu.sync_copy(o_v, o_hbm.at[pl.ds(off, blk)])
        return carry

    lax.fori_loop(0, nbatch, batch_body, 0)


def _tc_body(f_ref, b_ref, w_ref, o_ref, *, ks):
    f = f_ref[...]
    d = jnp.abs(f - b_ref[...])
    bits = lax.bitcast_convert_type(d, jnp.int32)
    rows = bits.shape[0]
    wt = jnp.zeros_like(f)
    for j, k in enumerate(ks):
        t = jnp.zeros((rows, 1), jnp.int32)
        for i in range(_NBITS):
            cand = t | jnp.int32(1 << (_NBITS - 1 - i))
            cnt = jnp.sum((bits >= cand).astype(jnp.int32), axis=1,
                          keepdims=True)
            t = jnp.where(cnt >= k, cand, t)
        wj = w_ref[0, j]
        wt = wt + jnp.where(bits >= t, wj, jnp.float32(0.0))
    o_ref[...] = f * wt


def kernel(fused_proto, base_proto, logits):
    q, n, d = fused_proto.shape
    r = q * n
    ks = tuple(max(1, int(ratio * d)) for ratio in _RATIOS)
    w = jax.nn.softmax(logits)
    wmat = jnp.broadcast_to(w[:, None], (3, 16)).reshape(-1)
    w_pad = jnp.zeros((8, 128), jnp.float32).at[0, :3].set(w)

    f2 = fused_proto.reshape(r, d)
    b2 = base_proto.reshape(r, d)
    f1 = fused_proto.reshape(-1)
    b1 = base_proto.reshape(-1)

    sc_rows = _SC_ROWS
    tc_rows = r - sc_rows

    mesh = plsc.VectorSubcoreMesh(core_axis_name="c", subcore_axis_name="s")
    sck = functools.partial(
        pl.kernel,
        mesh=mesh,
        out_type=jax.ShapeDtypeStruct((sc_rows * d,), jnp.float32),
        scratch_types=[
            pltpu.VMEM((_BR * _D,), jnp.float32),
            pltpu.VMEM((_BR * _D,), jnp.float32),
            pltpu.VMEM((_BR * _D,), jnp.float32),
            pltpu.VMEM((_D,), jnp.uint32),
            pltpu.VMEM((48,), jnp.float32),
        ],
        compiler_params=pltpu.CompilerParams(needs_layout_passes=False),
    )(functools.partial(_sc_body, sc_rows=sc_rows))
    out_sc = sck(f1, b1, wmat)

    br = 512
    grid = tc_rows // br
    blk0 = sc_rows // br
    out_tc = pl.pallas_call(
        functools.partial(_tc_body, ks=ks),
        grid=(grid,),
        in_specs=[
            pl.BlockSpec((br, d), lambda i: (i + blk0, 0)),
            pl.BlockSpec((br, d), lambda i: (i + blk0, 0)),
            pl.BlockSpec((8, 128), lambda i: (0, 0)),
        ],
        out_specs=pl.BlockSpec((br, d), lambda i: (i, 0)),
        out_shape=jax.ShapeDtypeStruct((tc_rows, d), jnp.float32),
        compiler_params=pltpu.CompilerParams(
            dimension_semantics=("arbitrary",),
        ),
    )(f2, b2, w_pad)

    out = jnp.concatenate([out_sc.reshape(sc_rows, d), out_tc], axis=0)
    return out.reshape(q, n, d)


# speedup vs baseline: 1.0280x; 1.0280x over previous
import functools

import jax
import jax.numpy as jnp
from jax import lax
from jax.experimental import pallas as pl
from jax.experimental.pallas import tpu as pltpu
from jax.experimental.pallas import tpu_sc as plsc

_RATIOS = (0.1, 0.25, 0.5)
_D = 1024
_NV = _D // 16
_BR = 32
_IMAX = 0xFFFFFFFF
_NW = 32
_SC_ROWS = 9216
_NBITS = 31


def _vsort(x, desc):
    k, _ = plsc.sort_key_val(x, x, descending=desc)
    return k


def _lane_min(v):
    return lax.reduce_min(v, (0,))


def _sc_body(f_hbm, b_hbm, w_hbm, o_hbm, f_v, b_v, o_v, bits_v, w_v, *,
             sc_rows):
    nc = 2
    wid = lax.axis_index("s") * nc + lax.axis_index("c")
    rows_per_w = sc_rows // _NW
    nbatch = rows_per_w // _BR
    blk = _BR * _D

    pltpu.sync_copy(w_hbm, w_v)
    w0 = w_v[pl.ds(0, 16)]
    w1 = w_v[pl.ds(16, 16)]
    w2 = w_v[pl.ds(32, 16)]

    def batch_body(t, carry):
        off = (wid * rows_per_w + t * _BR) * _D
        pltpu.sync_copy(f_hbm.at[pl.ds(off, blk)], f_v)
        pltpu.sync_copy(b_hbm.at[pl.ds(off, blk)], b_v)

        def row_body(r, carry2):
            boff = r * _D
            vs = []
            for v in range(_NV):
                fv = f_v[pl.ds(boff + v * 16, 16)]
                bv = b_v[pl.ds(boff + v * 16, 16)]
                dbits = lax.bitcast_convert_type(jnp.abs(fv - bv), jnp.uint32)
                bits_v[pl.ds(v * 16, 16)] = dbits
                vs.append(_vsort(dbits, desc=(v % 2 == 1)))
            for k in range(5, 10):
                sh = k - 4
                dv = 1 << (sh - 1)
                while dv >= 1:
                    for g in range(0, _NV, 2 * dv):
                        for i in range(g, g + dv):
                            asc = ((i >> sh) % 2 == 0)
                            a, b = vs[i], vs[i + dv]
                            lo = jnp.minimum(a, b)
                            hi = jnp.maximum(a, b)
                            vs[i], vs[i + dv] = (lo, hi) if asc else (hi, lo)
                    dv //= 2
                vs = [_vsort(x, desc=(((i >> sh) % 2) == 1))
                      for i, x in enumerate(vs)]
            u = [jnp.maximum(vs[i], vs[i + 32]) for i in range(32)]
            m = u[0]
            for x in u[1:]:
                m = jnp.minimum(m, x)
            t512 = jnp.full((16,), _lane_min(m), jnp.uint32)
            u2 = [jnp.maximum(u[i], u[i + 16]) for i in range(16)]
            m = u2[0]
            for x in u2[1:]:
                m = jnp.minimum(m, x)
            t256 = jnp.full((16,), _lane_min(m), jnp.uint32)
            u3 = [jnp.maximum(u2[i], u2[i + 8]) for i in range(8)]
            for dv in (4, 2, 1):
                for g in range(0, 8, 2 * dv):
                    for i in range(g, g + dv):
                        a, b = u3[i], u3[i + dv]
                        u3[i] = jnp.minimum(a, b)
                        u3[i + dv] = jnp.maximum(a, b)
            u3 = [_vsort(x, desc=False) for x in u3]
            lane = lax.broadcasted_iota(jnp.int32, (16,), 0)
            pick = jnp.where(lane == 10, u3[1],
                             jnp.full((16,), _IMAX, jnp.uint32))
            t102 = jnp.full((16,), _lane_min(pick), jnp.uint32)

            zero = jnp.zeros((16,), jnp.float32)
            for v in range(_NV):
                x = bits_v[pl.ds(v * 16, 16)]
                fv = f_v[pl.ds(boff + v * 16, 16)]
                wt = jnp.where(x >= t102, w0, zero)
                wt = wt + jnp.where(x >= t256, w1, zero)
                wt = wt + jnp.where(x >= t512, w2, zero)
                o_v[pl.ds(boff + v * 16, 16)] = fv * wt
            return carry2

        lax.fori_loop(0, _BR, row_body, 0)
        pltpu.sync_copy(o_v, o_hbm.at[pl.ds(off, blk)])
        return carry

    lax.fori_loop(0, nbatch, batch_body, 0)


def _tc_body(f_ref, b_ref, w_ref, o_ref, *, ks):
    f = f_ref[...]
    d = jnp.abs(f - b_ref[...])
    bits = lax.bitcast_convert_type(d, jnp.int32)
    rows = bits.shape[0]
    wt = jnp.zeros_like(f)
    for j, k in enumerate(ks):
        t = jnp.zeros((rows, 1), jnp.int32)
        for i in range(_NBITS):
            cand = t | jnp.int32(1 << (_NBITS - 1 - i))
            cnt = jnp.sum((bits >= cand).astype(jnp.int32), axis=1,
                          keepdims=True)
            t = jnp.where(cnt >= k, cand, t)
        wj = w_ref[0, j]
        wt = wt + jnp.where(bits >= t, wj, jnp.float32(0.0))
    o_ref[...] = f * wt


def kernel(fused_proto, base_proto, logits):
    q, n, d = fused_proto.shape
    r = q * n
    ks = tuple(max(1, int(ratio * d)) for ratio in _RATIOS)
    w = jax.nn.softmax(logits)
    wmat = jnp.broadcast_to(w[:, None], (3, 16)).reshape(-1)
    w_pad = jnp.zeros((8, 128), jnp.float32).at[0, :3].set(w)

    f2 = fused_proto.reshape(r, d)
    b2 = base_proto.reshape(r, d)
    f1 = fused_proto.reshape(-1)
    b1 = base_proto.reshape(-1)

    sc_rows = _SC_ROWS
    tc_rows = r - sc_rows

    mesh = plsc.VectorSubcoreMesh(core_axis_name="c", subcore_axis_name="s")
    sck = functools.partial(
        pl.kernel,
        mesh=mesh,
        out_type=jax.ShapeDtypeStruct((sc_rows * d,), jnp.float32),
        scratch_types=[
            pltpu.VMEM((_BR * _D,), jnp.float32),
            pltpu.VMEM((_BR * _D,), jnp.float32),
            pltpu.VMEM((_BR * _D,), jnp.float32),
            pltpu.VMEM((_D,), jnp.uint32),
            pltpu.VMEM((48,), jnp.float32),
        ],
        compiler_params=pltpu.CompilerParams(needs_layout_passes=False),
    )(functools.partial(_sc_body, sc_rows=sc_rows))
    out_sc = sck(f1, b1, wmat)

    br = 512
    grid = tc_rows // br
    blk0 = sc_rows // br
    out_tc = pl.pallas_call(
        functools.partial(_tc_body, ks=ks),
        grid=(grid,),
        in_specs=[
            pl.BlockSpec((br, d), lambda i: (i + blk0, 0)),
            pl.BlockSpec((br, d), lambda i: (i + blk0, 0)),
            pl.BlockSpec((8, 128), lambda i: (0, 0)),
        ],
        out_specs=pl.BlockSpec((br, d), lambda i: (i, 0)),
        out_shape=jax.ShapeDtypeStruct((tc_rows, d), jnp.float32),
        compiler_params=pltpu.CompilerParams(
            dimension_semantics=("arbitrary",),
        ),
    )(f2, b2, w_pad)

    out = jnp.concatenate([out_sc.reshape(sc_rows, d), out_tc], axis=0)
    return out.reshape(q, n, d)


# speedup vs baseline: 1.0336x; 1.0054x over previous
import functools

import jax
import jax.numpy as jnp
from jax import lax
from jax.experimental import pallas as pl
from jax.experimental.pallas import tpu as pltpu
from jax.experimental.pallas import tpu_sc as plsc

_RATIOS = (0.1, 0.25, 0.5)
_D = 1024
_NV = _D // 16
_BR = 48
_IMAX = 0xFFFFFFFF
_NW = 32
_SC_ROWS = 9216
_NBITS = 31


def _vsort(x, desc):
    k, _ = plsc.sort_key_val(x, x, descending=desc)
    return k


def _lane_min(v):
    return lax.reduce_min(v, (0,))


def _sc_body(f_hbm, b_hbm, w_hbm, o_hbm, f_v, b_v, bits_v, w_v, *,
             sc_rows):
    o_v = b_v
    nc = 2
    wid = lax.axis_index("s") * nc + lax.axis_index("c")
    rows_per_w = sc_rows // _NW
    nbatch = rows_per_w // _BR
    blk = _BR * _D

    pltpu.sync_copy(w_hbm, w_v)
    w0 = w_v[pl.ds(0, 16)]
    w1 = w_v[pl.ds(16, 16)]
    w2 = w_v[pl.ds(32, 16)]

    def batch_body(t, carry):
        off = (wid * rows_per_w + t * _BR) * _D
        pltpu.sync_copy(f_hbm.at[pl.ds(off, blk)], f_v)
        pltpu.sync_copy(b_hbm.at[pl.ds(off, blk)], b_v)

        def row_body(r, carry2):
            boff = r * _D
            vs = []
            for v in range(_NV):
                fv = f_v[pl.ds(boff + v * 16, 16)]
                bv = b_v[pl.ds(boff + v * 16, 16)]
                dbits = lax.bitcast_convert_type(jnp.abs(fv - bv), jnp.uint32)
                bits_v[pl.ds(v * 16, 16)] = dbits
                vs.append(_vsort(dbits, desc=(v % 2 == 1)))
            for k in range(5, 10):
                sh = k - 4
                dv = 1 << (sh - 1)
                while dv >= 1:
                    for g in range(0, _NV, 2 * dv):
                        for i in range(g, g + dv):
                            asc = ((i >> sh) % 2 == 0)
                            a, b = vs[i], vs[i + dv]
                            lo = jnp.minimum(a, b)
                            hi = jnp.maximum(a, b)
                            vs[i], vs[i + dv] = (lo, hi) if asc else (hi, lo)
                    dv //= 2
                vs = [_vsort(x, desc=(((i >> sh) % 2) == 1))
                      for i, x in enumerate(vs)]
            u = [jnp.maximum(vs[i], vs[i + 32]) for i in range(32)]
            m = u[0]
            for x in u[1:]:
                m = jnp.minimum(m, x)
            t512 = jnp.full((16,), _lane_min(m), jnp.uint32)
            u2 = [jnp.maximum(u[i], u[i + 16]) for i in range(16)]
            m = u2[0]
            for x in u2[1:]:
                m = jnp.minimum(m, x)
            t256 = jnp.full((16,), _lane_min(m), jnp.uint32)
            u3 = [jnp.maximum(u2[i], u2[i + 8]) for i in range(8)]
            for dv in (4, 2, 1):
                for g in range(0, 8, 2 * dv):
                    for i in range(g, g + dv):
                        a, b = u3[i], u3[i + dv]
                        u3[i] = jnp.minimum(a, b)
                        u3[i + dv] = jnp.maximum(a, b)
            u3 = [_vsort(x, desc=False) for x in u3]
            lane = lax.broadcasted_iota(jnp.int32, (16,), 0)
            pick = jnp.where(lane == 10, u3[1],
                             jnp.full((16,), _IMAX, jnp.uint32))
            t102 = jnp.full((16,), _lane_min(pick), jnp.uint32)

            zero = jnp.zeros((16,), jnp.float32)
            for v in range(_NV):
                x = bits_v[pl.ds(v * 16, 16)]
                fv = f_v[pl.ds(boff + v * 16, 16)]
                wt = jnp.where(x >= t102, w0, zero)
                wt = wt + jnp.where(x >= t256, w1, zero)
                wt = wt + jnp.where(x >= t512, w2, zero)
                o_v[pl.ds(boff + v * 16, 16)] = fv * wt
            return carry2

        lax.fori_loop(0, _BR, row_body, 0)
        pltpu.sync_copy(o_v, o_hbm.at[pl.ds(off, blk)])
        return carry

    lax.fori_loop(0, nbatch, batch_body, 0)


def _tc_body(f_ref, b_ref, w_ref, o_ref, *, ks):
    f = f_ref[...]
    d = jnp.abs(f - b_ref[...])
    bits = lax.bitcast_convert_type(d, jnp.int32)
    rows = bits.shape[0]
    wt = jnp.zeros_like(f)
    for j, k in enumerate(ks):
        t = jnp.zeros((rows, 1), jnp.int32)
        for i in range(_NBITS):
            cand = t | jnp.int32(1 << (_NBITS - 1 - i))
            cnt = jnp.sum((bits >= cand).astype(jnp.int32), axis=1,
                          keepdims=True)
            t = jnp.where(cnt >= k, cand, t)
        wj = w_ref[0, j]
        wt = wt + jnp.where(bits >= t, wj, jnp.float32(0.0))
    o_ref[...] = f * wt


def kernel(fused_proto, base_proto, logits):
    q, n, d = fused_proto.shape
    r = q * n
    ks = tuple(max(1, int(ratio * d)) for ratio in _RATIOS)
    w = jax.nn.softmax(logits)
    wmat = jnp.broadcast_to(w[:, None], (3, 16)).reshape(-1)
    w_pad = jnp.zeros((8, 128), jnp.float32).at[0, :3].set(w)

    f2 = fused_proto.reshape(r, d)
    b2 = base_proto.reshape(r, d)
    f1 = fused_proto.reshape(-1)
    b1 = base_proto.reshape(-1)

    sc_rows = _SC_ROWS
    tc_rows = r - sc_rows

    mesh = plsc.VectorSubcoreMesh(core_axis_name="c", subcore_axis_name="s")
    sck = functools.partial(
        pl.kernel,
        mesh=mesh,
        out_type=jax.ShapeDtypeStruct((sc_rows * d,), jnp.float32),
        scratch_types=[
            pltpu.VMEM((_BR * _D,), jnp.float32),
            pltpu.VMEM((_BR * _D,), jnp.float32),
            pltpu.VMEM((_D,), jnp.uint32),
            pltpu.VMEM((48,), jnp.float32),
        ],
        compiler_params=pltpu.CompilerParams(needs_layout_passes=False),
    )(functools.partial(_sc_body, sc_rows=sc_rows))
    out_sc = sck(f1, b1, wmat)

    br = 512
    grid = tc_rows // br
    blk0 = sc_rows // br
    out_tc = pl.pallas_call(
        functools.partial(_tc_body, ks=ks),
        grid=(grid,),
        in_specs=[
            pl.BlockSpec((br, d), lambda i: (i + blk0, 0)),
            pl.BlockSpec((br, d), lambda i: (i + blk0, 0)),
            pl.BlockSpec((8, 128), lambda i: (0, 0)),
        ],
        out_specs=pl.BlockSpec((br, d), lambda i: (i, 0)),
        out_shape=jax.ShapeDtypeStruct((tc_rows, d), jnp.float32),
        compiler_params=pltpu.CompilerParams(
            dimension_semantics=("arbitrary",),
        ),
    )(f2, b2, w_pad)

    out = jnp.concatenate([out_sc.reshape(sc_rows, d), out_tc], axis=0)
    return out.reshape(q, n, d)
